# baseline (speedup 1.0000x reference)
import functools

import jax
import jax.numpy as jnp
from jax import lax
from jax.experimental import pallas as pl
from jax.experimental.pallas import tpu as pltpu
from jax.experimental.pallas import tpu_sc as plsc

NC = 2
NS = 16
L = 16
NW = NC * NS

MAIN = 999936
TAIL = 1000000 - MAIN


def _rsqrt(x):
    i = plsc.bitcast(x, jnp.int32)
    i = jnp.int32(0x5F3759DF) - lax.shift_right_logical(i, 1)
    y = plsc.bitcast(i, jnp.float32)
    xh = x * jnp.float32(0.5)
    for _ in range(3):
        y = y * (jnp.float32(1.5) - xh * y * y)
    return y


CHUNK = (MAIN // (NW * 128)) * 128
LAST_EXTRA = MAIN - NW * CHUNK


def _make_sc_detile():
    mesh = plsc.VectorSubcoreMesh(
        core_axis_name="c", subcore_axis_name="s", num_cores=NC, num_subcores=NS
    )

    @functools.partial(
        pl.kernel,
        mesh=mesh,
        out_type=jax.ShapeDtypeStruct((1, 3 * MAIN), jnp.float32),
        scratch_types=[
            pltpu.SemaphoreType.DMA,
            pltpu.SemaphoreType.DMA,
            pltpu.SemaphoreType.DMA,
            pltpu.SemaphoreType.DMA,
        ],
        compiler_params=pltpu.CompilerParams(
            needs_layout_passes=False, use_tc_tiling_on_sc=True
        ),
    )
    def sc_detile(tt_hbm, out_hbm, s0, s1, s2, s3):
        wid = lax.axis_index("s") * NC + lax.axis_index("c")
        off = wid * CHUNK
        sems = (s0, s1, s2)
        cps = [
            pltpu.async_copy(
                tt_hbm.at[pl.ds(c, 1), pl.ds(off, CHUNK)],
                out_hbm.at[pl.ds(0, 1), pl.ds(c * MAIN + off, CHUNK)],
                sems[c],
            )
            for c in range(3)
        ]

        @pl.when(wid == NW - 1)
        def _():
            tail_off = NW * CHUNK
            for c in range(3):
                pltpu.async_copy(
                    tt_hbm.at[pl.ds(c, 1), pl.ds(tail_off, LAST_EXTRA)],
                    out_hbm.at[
                        pl.ds(0, 1), pl.ds(c * MAIN + tail_off, LAST_EXTRA)
                    ],
                    s3,
                ).wait()

        for cp in cps:
            cp.wait()

    return sc_detile


def _make_sc_kernel(batch):
    b_per_w = batch // NW
    mesh = plsc.VectorSubcoreMesh(
        core_axis_name="c", subcore_axis_name="s", num_cores=NC, num_subcores=NS
    )

    @functools.partial(
        pl.kernel,
        mesh=mesh,
        out_type=jax.ShapeDtypeStruct((9, batch), jnp.float32),
        scratch_types=[
            pltpu.VMEM((b_per_w,), jnp.int32),
            pltpu.VMEM((b_per_w,), jnp.int32),
            pltpu.VMEM((b_per_w,), jnp.int32),
            pltpu.VMEM((b_per_w,), jnp.float32),
            pltpu.VMEM((b_per_w,), jnp.float32),
            pltpu.VMEM((b_per_w,), jnp.float32),
            pltpu.VMEM((3 * TAIL,), jnp.float32),
            pltpu.VMEM((9, b_per_w), jnp.float32),
            pltpu.SemaphoreType.DMA,
            pltpu.SemaphoreType.DMA,
            pltpu.SemaphoreType.DMA,
        ],
        compiler_params=pltpu.CompilerParams(
            needs_layout_passes=False, use_tc_tiling_on_sc=True
        ),
    )
    def sc_kernel(tflat2_hbm, tail_hbm, idx_hbm, out_hbm,
                  idx0_v, idx1_v, idx2_v, l0_v, l1_v, l2_v, tail_v, out9_v,
                  s0, s1, s2):
        tflat_hbm = tflat2_hbm.at[0]
        wid = lax.axis_index("s") * NC + lax.axis_index("c")
        base = wid * b_per_w
        pltpu.sync_copy(idx_hbm.at[pl.ds(base, b_per_w)], idx0_v)
        pltpu.sync_copy(tail_hbm, tail_v)
        zero_i = jnp.zeros((L,), jnp.int32)
        for g in range(b_per_w // L):
            sl = pl.ds(g * L, L)
            i0 = idx0_v[sl]
            safe = jnp.where(i0 < jnp.int32(MAIN), i0, zero_i)
            idx0_v[sl] = safe
            idx1_v[sl] = safe + jnp.int32(MAIN)
            idx2_v[sl] = safe + jnp.int32(2 * MAIN)
        cp0 = pltpu.async_copy(tflat_hbm.at[idx0_v], l0_v, s0)
        cp1 = pltpu.async_copy(tflat_hbm.at[idx1_v], l1_v, s1)
        cp2 = pltpu.async_copy(tflat_hbm.at[idx2_v], l2_v, s2)
        cp0.wait()
        cp1.wait()
        cp2.wait()
        pltpu.sync_copy(idx_hbm.at[pl.ds(base, b_per_w)], idx0_v)

        zero_f = jnp.zeros((L,), jnp.float32)
        for g in range(b_per_w // L):
            sl = pl.ds(g * L, L)
            i0 = idx0_v[sl]
            in_tail = i0 >= jnp.int32(MAIN)
            it = jnp.where(in_tail, i0 - jnp.int32(MAIN), zero_i)
            l0 = jnp.where(
                in_tail, plsc.load_gather(tail_v, [it]), l0_v[sl])
            l1 = jnp.where(
                in_tail, plsc.load_gather(tail_v, [it + jnp.int32(TAIL)]),
                l1_v[sl])
            l2 = jnp.where(
                in_tail, plsc.load_gather(tail_v, [it + jnp.int32(2 * TAIL)]),
                l2_v[sl])

            s = l0 * l0 + l1 * l1
            r2 = _rsqrt(s)
            r3 = _rsqrt(s + l2 * l2)
            q = r2 * r3
            t = l2 * q
            out9_v[0, sl] = l1 * r2
            out9_v[1, sl] = -(l0 * t)
            out9_v[2, sl] = -(l0 * r3)
            out9_v[3, sl] = -(l0 * r2)
            out9_v[4, sl] = -(l1 * t)
            out9_v[5, sl] = -(l1 * r3)
            out9_v[6, sl] = zero_f
            out9_v[7, sl] = s * q
            out9_v[8, sl] = -(l2 * r3)

        pltpu.sync_copy(out9_v, out_hbm.at[:, pl.ds(base, b_per_w)])

    return sc_kernel


@jax.jit
def kernel(idx, focal_length, principal_point, T, table):
    batch = idx.shape[0]
    tt = table.T
    tflat = _make_sc_detile()(tt)
    tail = tt[:, MAIN:].reshape(3 * TAIL)
    out9 = _make_sc_kernel(batch)(tflat, tail, idx)
    rotmat = jnp.transpose(out9.reshape(3, 3, 1, batch), (2, 3, 0, 1))
    return (rotmat, focal_length, principal_point, T)


# speedup vs baseline: 6.5400x; 6.5400x over previous
import functools

import jax
import jax.numpy as jnp
from jax import lax
from jax.experimental import pallas as pl
from jax.experimental.pallas import tpu as pltpu
from jax.experimental.pallas import tpu_sc as plsc

NC = 2
NS = 16
L = 16
NW = NC * NS


def _rsqrt(x):
    i = plsc.bitcast(x, jnp.int32)
    i = jnp.int32(0x5F3759DF) - lax.shift_right_logical(i, 1)
    y = plsc.bitcast(i, jnp.float32)
    xh = x * jnp.float32(0.5)
    for _ in range(3):
        y = y * (jnp.float32(1.5) - xh * y * y)
    return y


def _make_sc_kernel(batch):
    b_per_w = batch // NW
    mesh = plsc.VectorSubcoreMesh(
        core_axis_name="c", subcore_axis_name="s", num_cores=NC, num_subcores=NS
    )

    @functools.partial(
        pl.kernel,
        mesh=mesh,
        out_type=jax.ShapeDtypeStruct((9, batch), jnp.float32),
        scratch_types=[
            pltpu.VMEM((b_per_w,), jnp.int32),
            pltpu.VMEM((b_per_w,), jnp.float32),
            pltpu.VMEM((b_per_w,), jnp.float32),
            pltpu.VMEM((b_per_w,), jnp.float32),
            pltpu.VMEM((9, b_per_w), jnp.float32),
            pltpu.SemaphoreType.DMA,
            pltpu.SemaphoreType.DMA,
            pltpu.SemaphoreType.DMA,
        ],
        compiler_params=pltpu.CompilerParams(
            needs_layout_passes=False, use_tc_tiling_on_sc=False
        ),
    )
    def sc_kernel(tt_hbm, idx_hbm, out_hbm,
                  idx_v, l0_v, l1_v, l2_v, out9_v, s0, s1, s2):
        wid = lax.axis_index("s") * NC + lax.axis_index("c")
        base = wid * b_per_w
        pltpu.sync_copy(idx_hbm.at[pl.ds(base, b_per_w)], idx_v)
        cp0 = pltpu.async_copy(tt_hbm.at[0].at[idx_v], l0_v, s0)
        cp1 = pltpu.async_copy(tt_hbm.at[1].at[idx_v], l1_v, s1)
        cp2 = pltpu.async_copy(tt_hbm.at[2].at[idx_v], l2_v, s2)
        cp0.wait()
        cp1.wait()
        cp2.wait()

        zero_f = jnp.zeros((L,), jnp.float32)
        for g in range(b_per_w // L):
            sl = pl.ds(g * L, L)
            l0 = l0_v[sl]
            l1 = l1_v[sl]
            l2 = l2_v[sl]

            s = l0 * l0 + l1 * l1
            r2 = _rsqrt(s)
            r3 = _rsqrt(s + l2 * l2)
            q = r2 * r3
            t = l2 * q
            out9_v[0, sl] = l1 * r2
            out9_v[1, sl] = -(l0 * t)
            out9_v[2, sl] = -(l0 * r3)
            out9_v[3, sl] = -(l0 * r2)
            out9_v[4, sl] = -(l1 * t)
            out9_v[5, sl] = -(l1 * r3)
            out9_v[6, sl] = zero_f
            out9_v[7, sl] = s * q
            out9_v[8, sl] = -(l2 * r3)

        pltpu.sync_copy(out9_v, out_hbm.at[:, pl.ds(base, b_per_w)])

    return sc_kernel


@jax.jit
def kernel(idx, focal_length, principal_point, T, table):
    batch = idx.shape[0]
    out9 = _make_sc_kernel(batch)(table.T, idx)
    rotmat = jnp.transpose(out9.reshape(3, 3, 1, batch), (2, 3, 0, 1))
    return (rotmat, focal_length, principal_point, T)


# speedup vs baseline: 6.8681x; 1.0502x over previous
import functools

import jax
import jax.numpy as jnp
from jax import lax
from jax.experimental import pallas as pl
from jax.experimental.pallas import tpu as pltpu
from jax.experimental.pallas import tpu_sc as plsc

NC = 2
NS = 16
L = 16
NW = NC * NS


def _rsqrt(x):
    i = plsc.bitcast(x, jnp.int32)
    i = jnp.int32(0x5F3759DF) - lax.shift_right_logical(i, 1)
    y = plsc.bitcast(i, jnp.float32)
    xh = x * jnp.float32(0.5)
    for _ in range(3):
        y = y * (jnp.float32(1.5) - xh * y * y)
    return y


def _make_sc_kernel(batch):
    b_per_w = batch // NW
    mesh = plsc.VectorSubcoreMesh(
        core_axis_name="c", subcore_axis_name="s", num_cores=NC, num_subcores=NS
    )

    @functools.partial(
        pl.kernel,
        mesh=mesh,
        out_type=jax.ShapeDtypeStruct((3, 3, 1, batch), jnp.float32),
        scratch_types=[
            pltpu.VMEM((b_per_w,), jnp.int32),
            pltpu.VMEM((b_per_w,), jnp.float32),
            pltpu.VMEM((b_per_w,), jnp.float32),
            pltpu.VMEM((b_per_w,), jnp.float32),
            pltpu.VMEM((3, 3, 1, b_per_w), jnp.float32),
            pltpu.SemaphoreType.DMA,
            pltpu.SemaphoreType.DMA,
            pltpu.SemaphoreType.DMA,
        ],
        compiler_params=pltpu.CompilerParams(
            needs_layout_passes=False, use_tc_tiling_on_sc=False
        ),
    )
    def sc_kernel(tt_hbm, idx_hbm, out_hbm,
                  idx_v, l0_v, l1_v, l2_v, out9_v, s0, s1, s2):
        wid = lax.axis_index("s") * NC + lax.axis_index("c")
        base = wid * b_per_w
        pltpu.sync_copy(idx_hbm.at[pl.ds(base, b_per_w)], idx_v)
        cp0 = pltpu.async_copy(tt_hbm.at[0].at[idx_v], l0_v, s0)
        cp1 = pltpu.async_copy(tt_hbm.at[1].at[idx_v], l1_v, s1)
        cp2 = pltpu.async_copy(tt_hbm.at[2].at[idx_v], l2_v, s2)
        cp0.wait()
        cp1.wait()
        cp2.wait()

        zero_f = jnp.zeros((L,), jnp.float32)
        for g in range(b_per_w // L):
            sl = pl.ds(g * L, L)
            l0 = l0_v[sl]
            l1 = l1_v[sl]
            l2 = l2_v[sl]

            s = l0 * l0 + l1 * l1
            r2 = _rsqrt(s)
            r3 = _rsqrt(s + l2 * l2)
            q = r2 * r3
            t = l2 * q
            out9_v[0, 0, 0, sl] = l1 * r2
            out9_v[0, 1, 0, sl] = -(l0 * t)
            out9_v[0, 2, 0, sl] = -(l0 * r3)
            out9_v[1, 0, 0, sl] = -(l0 * r2)
            out9_v[1, 1, 0, sl] = -(l1 * t)
            out9_v[1, 2, 0, sl] = -(l1 * r3)
            out9_v[2, 0, 0, sl] = zero_f
            out9_v[2, 1, 0, sl] = s * q
            out9_v[2, 2, 0, sl] = -(l2 * r3)

        pltpu.sync_copy(
            out9_v, out_hbm.at[:, :, :, pl.ds(base, b_per_w)])

    return sc_kernel


@jax.jit
def kernel(idx, focal_length, principal_point, T, table):
    batch = idx.shape[0]
    out9 = _make_sc_kernel(batch)(table.T, idx)
    rotmat = jnp.transpose(out9, (2, 3, 0, 1))
    return (rotmat, focal_length, principal_point, T)
